# baseline (speedup 1.0000x reference)
import jax
import jax.numpy as jnp
from jax import lax
from jax.experimental import pallas as pl
from jax.experimental.pallas import tpu as pltpu
from jax.experimental.pallas import tpu_sc as plsc

N = 10000
E = 320000
D = 128
C = 10
G = 64

NC = 2
NS = 16
NW = NC * NS
N_PAD = 10240
ROWS_PER_TILE = N_PAD // NS
K = 128
CHP = 80
EPT = CHP * K
NBUF = 4

_mesh = plsc.VectorSubcoreMesh(core_axis_name="c", subcore_axis_name="s")


def _sc_deg_body(dst_hbm, out_hbm, didx, ones_v, zbuf, acc, s0, s1, s2, s3):
    c = lax.axis_index("c")
    s = lax.axis_index("s")
    wid = c * NS + s
    ssem = (s0, s1, s2, s3)

    @pl.loop(0, K, step=16)
    def _(i):
        ones_v[pl.ds(i, 16)] = jnp.ones((16,), jnp.float32)

    @pl.loop(0, ROWS_PER_TILE, step=16)
    def _(i):
        zbuf[pl.ds(i, 16)] = jnp.zeros((16,), jnp.float32)

    pltpu.sync_copy(zbuf, acc.at[pl.ds(s * ROWS_PER_TILE, ROWS_PER_TILE)])
    pltpu.sync_copy(dst_hbm.at[wid], didx)
    plsc.subcore_barrier()

    for b in range(NBUF):
        pltpu.async_copy(ones_v, acc.at[didx.at[b]], ssem[b], add=True)

    @pl.loop(0, CHP - NBUF, step=NBUF)
    def _(i):
        for b in range(NBUF):
            pltpu.make_async_copy(ones_v, acc.at[didx.at[0]], ssem[b]).wait()
            pltpu.async_copy(ones_v, acc.at[didx.at[i + NBUF + b]], ssem[b],
                             add=True)

    for b in range(NBUF):
        pltpu.make_async_copy(ones_v, acc.at[didx.at[0]], ssem[b]).wait()

    plsc.subcore_barrier()
    sl = pl.ds(s * ROWS_PER_TILE, ROWS_PER_TILE)
    pltpu.sync_copy(acc.at[sl], out_hbm.at[c, sl])


@jax.jit
def _sc_deg(dst_r):
    return pl.kernel(
        _sc_deg_body,
        out_type=jax.ShapeDtypeStruct((NC, N_PAD), jnp.float32),
        mesh=_mesh,
        scratch_types=[
            pltpu.VMEM((CHP, K), jnp.int32),
            pltpu.VMEM((K,), jnp.float32),
            pltpu.VMEM((ROWS_PER_TILE,), jnp.float32),
            pltpu.VMEM_SHARED((N_PAD,), jnp.float32),
            pltpu.SemaphoreType.DMA,
            pltpu.SemaphoreType.DMA,
            pltpu.SemaphoreType.DMA,
            pltpu.SemaphoreType.DMA,
        ],
    )(dst_r)


EK = 80
ECH = E // NW // EK


def _sc_edge_body(y_hbm, sd_hbm, out_hbm, sd, sd2,
                  rows, rows2, acc, gsem, gsem2):
    c = lax.axis_index("c")
    s = lax.axis_index("s")
    wid = c * NS + s

    @pl.loop(0, EK)
    def _(r):
        @pl.loop(0, D, step=16)
        def _(j):
            rows[r, pl.ds(j, 16)] = jnp.zeros((16,), jnp.float32)

    @pl.loop(0, ROWS_PER_TILE, step=EK)
    def _(q):
        pltpu.sync_copy(rows, acc.at[pl.ds(s * ROWS_PER_TILE + q, EK)])

    plsc.subcore_barrier()

    pltpu.sync_copy(sd_hbm.at[wid, 0], sd)
    pltpu.async_copy(y_hbm.at[sd.at[0]], rows, gsem)

    @pl.loop(0, ECH - 1, step=2)
    def _(i):
        pltpu.sync_copy(sd_hbm.at[wid, i + 1], sd2)
        pltpu.async_copy(y_hbm.at[sd2.at[0]], rows2, gsem2)
        pltpu.make_async_copy(y_hbm.at[sd.at[0]], rows, gsem).wait()
        pltpu.sync_copy(rows, acc.at[sd.at[1]], add=True)
        pltpu.sync_copy(sd_hbm.at[wid, i + 2], sd)
        pltpu.async_copy(y_hbm.at[sd.at[0]], rows, gsem)
        pltpu.make_async_copy(y_hbm.at[sd2.at[0]], rows2, gsem2).wait()
        pltpu.sync_copy(rows2, acc.at[sd2.at[1]], add=True)

    pltpu.make_async_copy(y_hbm.at[sd.at[0]], rows, gsem).wait()
    pltpu.sync_copy(rows, acc.at[sd.at[1]], add=True)

    plsc.subcore_barrier()
    sl = pl.ds(s * ROWS_PER_TILE, ROWS_PER_TILE)
    pltpu.sync_copy(acc.at[sl], out_hbm.at[c, sl])


@jax.jit
def _sc_edge(y, sd_e):
    return pl.kernel(
        _sc_edge_body,
        out_type=jax.ShapeDtypeStruct((NC, N_PAD, D), jnp.float32),
        mesh=_mesh,
        scratch_types=[
            pltpu.VMEM((2, EK), jnp.int32),
            pltpu.VMEM((2, EK), jnp.int32),
            pltpu.VMEM((EK, D), jnp.float32),
            pltpu.VMEM((EK, D), jnp.float32),
            pltpu.VMEM_SHARED((N_PAD, D), jnp.float32),
            pltpu.SemaphoreType.DMA,
            pltpu.SemaphoreType.DMA,
        ],
    )(y, sd_e)


_BR = 1024


def _tc_pre_body(deg0_ref, deg1_ref, x_ref, w_ref, dinv_ref, y_ref):
    deg = deg0_ref[...] + deg1_ref[...] + 1.0
    dinv = lax.rsqrt(deg)
    dinv_ref[...] = dinv
    y_ref[...] = jnp.dot(x_ref[...], w_ref[...],
                         preferred_element_type=jnp.float32) * dinv


@jax.jit
def _tc_pre(deg0, deg1, x, w):
    grid = (N_PAD // _BR,)
    return pl.pallas_call(
        _tc_pre_body,
        grid=grid,
        in_specs=[
            pl.BlockSpec((_BR, 1), lambda i: (i, 0)),
            pl.BlockSpec((_BR, 1), lambda i: (i, 0)),
            pl.BlockSpec((_BR, D), lambda i: (i, 0)),
            pl.BlockSpec((D, D), lambda i: (0, 0)),
        ],
        out_specs=[
            pl.BlockSpec((_BR, 1), lambda i: (i, 0)),
            pl.BlockSpec((_BR, D), lambda i: (i, 0)),
        ],
        out_shape=[
            jax.ShapeDtypeStruct((N_PAD, 1), jnp.float32),
            jax.ShapeDtypeStruct((N_PAD, D), jnp.float32),
        ],
    )(deg0, deg1, x, w)


def _tc_mid_body(a0_ref, a1_ref, y_ref, dinv_ref, b_ref, w_ref, o_ref):
    dinv = dinv_ref[...]
    h = dinv * (a0_ref[...] + a1_ref[...] + y_ref[...]) + b_ref[...]
    h = jnp.maximum(h, 0.0)
    o_ref[...] = jnp.dot(h, w_ref[...],
                         preferred_element_type=jnp.float32) * dinv


@jax.jit
def _tc_mid(a0, a1, y, dinv, b, w):
    grid = (N_PAD // _BR,)
    return pl.pallas_call(
        _tc_mid_body,
        grid=grid,
        in_specs=[
            pl.BlockSpec((_BR, D), lambda i: (i, 0)),
            pl.BlockSpec((_BR, D), lambda i: (i, 0)),
            pl.BlockSpec((_BR, D), lambda i: (i, 0)),
            pl.BlockSpec((_BR, 1), lambda i: (i, 0)),
            pl.BlockSpec((1, D), lambda i: (0, 0)),
            pl.BlockSpec((D, D), lambda i: (0, 0)),
        ],
        out_specs=pl.BlockSpec((_BR, D), lambda i: (i, 0)),
        out_shape=jax.ShapeDtypeStruct((N_PAD, D), jnp.float32),
    )(a0, a1, y, dinv, b, w)


_BRP = 512


def _tc_post_body(a0_ref, a1_ref, y_ref, dinv_ref, b_ref, batch_ref,
                  linw_ref, linb_ref, o_ref, pool_ref, cnt_ref):
    i = pl.program_id(0)

    @pl.when(i == 0)
    def _():
        pool_ref[...] = jnp.zeros_like(pool_ref)
        cnt_ref[...] = jnp.zeros_like(cnt_ref)

    h = dinv_ref[...] * (a0_ref[...] + a1_ref[...] + y_ref[...]) + b_ref[...]
    gids = lax.broadcasted_iota(jnp.int32, (G, _BRP), 0)
    onehot = (batch_ref[...] == gids).astype(jnp.float32)
    pool_ref[...] += lax.dot_general(
        onehot, h, (((1,), (0,)), ((), ())),
        preferred_element_type=jnp.float32)
    cnt_ref[...] += lax.dot_general(
        onehot, jnp.ones((_BRP, 1), jnp.float32), (((1,), (0,)), ((), ())),
        preferred_element_type=jnp.float32)

    @pl.when(i == (N_PAD // _BRP) - 1)
    def _():
        pooled = pool_ref[...] / jnp.maximum(cnt_ref[...], 1.0)
        o_ref[...] = jnp.dot(pooled, linw_ref[...],
                             preferred_element_type=jnp.float32) + linb_ref[...]


@jax.jit
def _tc_post(a0, a1, y, dinv, b, batch2d, lin_W, lin_b):
    grid = (N_PAD // _BRP,)
    return pl.pallas_call(
        _tc_post_body,
        grid=grid,
        in_specs=[
            pl.BlockSpec((_BRP, D), lambda i: (i, 0)),
            pl.BlockSpec((_BRP, D), lambda i: (i, 0)),
            pl.BlockSpec((_BRP, D), lambda i: (i, 0)),
            pl.BlockSpec((_BRP, 1), lambda i: (i, 0)),
            pl.BlockSpec((1, D), lambda i: (0, 0)),
            pl.BlockSpec((1, _BRP), lambda i: (0, i)),
            pl.BlockSpec((D, C), lambda i: (0, 0)),
            pl.BlockSpec((1, C), lambda i: (0, 0)),
        ],
        out_specs=pl.BlockSpec((G, C), lambda i: (0, 0)),
        out_shape=jax.ShapeDtypeStruct((G, C), jnp.float32),
        scratch_shapes=[
            pltpu.VMEM((G, D), jnp.float32),
            pltpu.VMEM((G, 1), jnp.float32),
        ],
    )(a0, a1, y, dinv, b, batch2d, lin_W, lin_b)


def kernel(x, edge_index, batch, W1, b1, W2, b2, W3, b3, lin_W, lin_b):
    pad_dst = jnp.broadcast_to(
        N + jnp.arange(EPT - E // NW, dtype=jnp.int32)[None, :],
        (NW, EPT - E // NW))
    dst_r = jnp.concatenate(
        [edge_index[1].astype(jnp.int32).reshape(NW, E // NW), pad_dst],
        axis=1).reshape(NW, CHP, K)
    sd_e = jnp.stack([edge_index[0].astype(jnp.int32).reshape(NW, ECH, EK),
                      edge_index[1].astype(jnp.int32).reshape(NW, ECH, EK)],
                     axis=2)
    x_pad = jnp.pad(x, ((0, N_PAD - N), (0, 0)))
    batch2d = jnp.pad(batch.astype(jnp.int32), (0, N_PAD - N),
                      constant_values=G).reshape(1, N_PAD)

    degp = _sc_deg(dst_r)
    deg0 = degp[0].reshape(N_PAD, 1)
    deg1 = degp[1].reshape(N_PAD, 1)
    dinv, y = _tc_pre(deg0, deg1, x_pad, W1)

    accp = _sc_edge(y, sd_e)
    y = _tc_mid(accp[0], accp[1], y, dinv, b1.reshape(1, D), W2)
    accp = _sc_edge(y, sd_e)
    y = _tc_mid(accp[0], accp[1], y, dinv, b2.reshape(1, D), W3)
    accp = _sc_edge(y, sd_e)
    return _tc_post(accp[0], accp[1], y, dinv, b3.reshape(1, D), batch2d,
                    lin_W, lin_b.reshape(1, C))


# speedup vs baseline: 1.0005x; 1.0005x over previous
import jax
import jax.numpy as jnp
from jax import lax
from jax.experimental import pallas as pl
from jax.experimental.pallas import tpu as pltpu
from jax.experimental.pallas import tpu_sc as plsc

N = 10000
E = 320000
D = 128
C = 10
G = 64

NC = 2
NS = 16
NW = NC * NS
N_PAD = 10240
ROWS_PER_TILE = N_PAD // NS
K = 128
CHP = 80
EPT = CHP * K
NBUF = 4

def _mesh():
    return plsc.VectorSubcoreMesh(core_axis_name="c", subcore_axis_name="s")


def _sc_deg_body(dst_hbm, out_hbm, didx, ones_v, zbuf, acc, s0, s1, s2, s3):
    c = lax.axis_index("c")
    s = lax.axis_index("s")
    wid = c * NS + s
    ssem = (s0, s1, s2, s3)

    @pl.loop(0, K, step=16)
    def _(i):
        ones_v[pl.ds(i, 16)] = jnp.ones((16,), jnp.float32)

    @pl.loop(0, ROWS_PER_TILE, step=16)
    def _(i):
        zbuf[pl.ds(i, 16)] = jnp.zeros((16,), jnp.float32)

    pltpu.sync_copy(zbuf, acc.at[pl.ds(s * ROWS_PER_TILE, ROWS_PER_TILE)])
    pltpu.sync_copy(dst_hbm.at[wid], didx)
    plsc.subcore_barrier()

    for b in range(NBUF):
        pltpu.async_copy(ones_v, acc.at[didx.at[b]], ssem[b], add=True)

    @pl.loop(0, CHP - NBUF, step=NBUF)
    def _(i):
        for b in range(NBUF):
            pltpu.make_async_copy(ones_v, acc.at[didx.at[0]], ssem[b]).wait()
            pltpu.async_copy(ones_v, acc.at[didx.at[i + NBUF + b]], ssem[b],
                             add=True)

    for b in range(NBUF):
        pltpu.make_async_copy(ones_v, acc.at[didx.at[0]], ssem[b]).wait()

    plsc.subcore_barrier()
    sl = pl.ds(s * ROWS_PER_TILE, ROWS_PER_TILE)
    pltpu.sync_copy(acc.at[sl], out_hbm.at[c, sl])


@jax.jit
def _sc_deg(dst_r):
    return pl.kernel(
        _sc_deg_body,
        out_type=jax.ShapeDtypeStruct((NC, N_PAD), jnp.float32),
        mesh=_mesh(),
        scratch_types=[
            pltpu.VMEM((CHP, K), jnp.int32),
            pltpu.VMEM((K,), jnp.float32),
            pltpu.VMEM((ROWS_PER_TILE,), jnp.float32),
            pltpu.VMEM_SHARED((N_PAD,), jnp.float32),
            pltpu.SemaphoreType.DMA,
            pltpu.SemaphoreType.DMA,
            pltpu.SemaphoreType.DMA,
            pltpu.SemaphoreType.DMA,
        ],
    )(dst_r)


EK = 80
ECH = E // NW // EK


def _sc_edge_body(y_hbm, sd_hbm, out_hbm, sd, sd2,
                  rows, rows2, acc, gsem, gsem2):
    c = lax.axis_index("c")
    s = lax.axis_index("s")
    wid = c * NS + s

    @pl.loop(0, EK)
    def _(r):
        @pl.loop(0, D, step=16)
        def _(j):
            rows[r, pl.ds(j, 16)] = jnp.zeros((16,), jnp.float32)

    @pl.loop(0, ROWS_PER_TILE, step=EK)
    def _(q):
        pltpu.sync_copy(rows, acc.at[pl.ds(s * ROWS_PER_TILE + q, EK)])

    plsc.subcore_barrier()

    pltpu.sync_copy(sd_hbm.at[wid, 0], sd)
    pltpu.async_copy(y_hbm.at[sd.at[0]], rows, gsem)

    @pl.loop(0, ECH - 1, step=2)
    def _(i):
        pltpu.sync_copy(sd_hbm.at[wid, i + 1], sd2)
        pltpu.async_copy(y_hbm.at[sd2.at[0]], rows2, gsem2)
        pltpu.make_async_copy(y_hbm.at[sd.at[0]], rows, gsem).wait()
        pltpu.sync_copy(rows, acc.at[sd.at[1]], add=True)
        pltpu.sync_copy(sd_hbm.at[wid, i + 2], sd)
        pltpu.async_copy(y_hbm.at[sd.at[0]], rows, gsem)
        pltpu.make_async_copy(y_hbm.at[sd2.at[0]], rows2, gsem2).wait()
        pltpu.sync_copy(rows2, acc.at[sd2.at[1]], add=True)

    pltpu.make_async_copy(y_hbm.at[sd.at[0]], rows, gsem).wait()
    pltpu.sync_copy(rows, acc.at[sd.at[1]], add=True)

    plsc.subcore_barrier()
    sl = pl.ds(s * ROWS_PER_TILE, ROWS_PER_TILE)
    pltpu.sync_copy(acc.at[sl], out_hbm.at[c, sl])


@jax.jit
def _sc_edge(y, sd_e):
    return pl.kernel(
        _sc_edge_body,
        out_type=jax.ShapeDtypeStruct((NC, N_PAD, D), jnp.float32),
        mesh=_mesh(),
        scratch_types=[
            pltpu.VMEM((2, EK), jnp.int32),
            pltpu.VMEM((2, EK), jnp.int32),
            pltpu.VMEM((EK, D), jnp.float32),
            pltpu.VMEM((EK, D), jnp.float32),
            pltpu.VMEM_SHARED((N_PAD, D), jnp.float32),
            pltpu.SemaphoreType.DMA,
            pltpu.SemaphoreType.DMA,
        ],
    )(y, sd_e)


_BR = 1024


def _tc_pre_body(deg0_ref, deg1_ref, x_ref, w_ref, dinv_ref, y_ref):
    deg = deg0_ref[...] + deg1_ref[...] + 1.0
    dinv = lax.rsqrt(deg)
    dinv_ref[...] = dinv
    y_ref[...] = jnp.dot(x_ref[...], w_ref[...],
                         preferred_element_type=jnp.float32) * dinv


@jax.jit
def _tc_pre(deg0, deg1, x, w):
    grid = (N_PAD // _BR,)
    return pl.pallas_call(
        _tc_pre_body,
        grid=grid,
        in_specs=[
            pl.BlockSpec((_BR, 1), lambda i: (i, 0)),
            pl.BlockSpec((_BR, 1), lambda i: (i, 0)),
            pl.BlockSpec((_BR, D), lambda i: (i, 0)),
            pl.BlockSpec((D, D), lambda i: (0, 0)),
        ],
        out_specs=[
            pl.BlockSpec((_BR, 1), lambda i: (i, 0)),
            pl.BlockSpec((_BR, D), lambda i: (i, 0)),
        ],
        out_shape=[
            jax.ShapeDtypeStruct((N_PAD, 1), jnp.float32),
            jax.ShapeDtypeStruct((N_PAD, D), jnp.float32),
        ],
    )(deg0, deg1, x, w)


def _tc_mid_body(a0_ref, a1_ref, y_ref, dinv_ref, b_ref, w_ref, o_ref):
    dinv = dinv_ref[...]
    h = dinv * (a0_ref[...] + a1_ref[...] + y_ref[...]) + b_ref[...]
    h = jnp.maximum(h, 0.0)
    o_ref[...] = jnp.dot(h, w_ref[...],
                         preferred_element_type=jnp.float32) * dinv


@jax.jit
def _tc_mid(a0, a1, y, dinv, b, w):
    grid = (N_PAD // _BR,)
    return pl.pallas_call(
        _tc_mid_body,
        grid=grid,
        in_specs=[
            pl.BlockSpec((_BR, D), lambda i: (i, 0)),
            pl.BlockSpec((_BR, D), lambda i: (i, 0)),
            pl.BlockSpec((_BR, D), lambda i: (i, 0)),
            pl.BlockSpec((_BR, 1), lambda i: (i, 0)),
            pl.BlockSpec((1, D), lambda i: (0, 0)),
            pl.BlockSpec((D, D), lambda i: (0, 0)),
        ],
        out_specs=pl.BlockSpec((_BR, D), lambda i: (i, 0)),
        out_shape=jax.ShapeDtypeStruct((N_PAD, D), jnp.float32),
    )(a0, a1, y, dinv, b, w)


_BRP = 512


def _tc_post_body(a0_ref, a1_ref, y_ref, dinv_ref, b_ref, batch_ref,
                  linw_ref, linb_ref, o_ref, pool_ref, cnt_ref):
    i = pl.program_id(0)

    @pl.when(i == 0)
    def _():
        pool_ref[...] = jnp.zeros_like(pool_ref)
        cnt_ref[...] = jnp.zeros_like(cnt_ref)

    h = dinv_ref[...] * (a0_ref[...] + a1_ref[...] + y_ref[...]) + b_ref[...]
    gids = lax.broadcasted_iota(jnp.int32, (G, _BRP), 0)
    onehot = (batch_ref[...] == gids).astype(jnp.float32)
    pool_ref[...] += lax.dot_general(
        onehot, h, (((1,), (0,)), ((), ())),
        preferred_element_type=jnp.float32)
    cnt_ref[...] += lax.dot_general(
        onehot, jnp.ones((_BRP, 1), jnp.float32), (((1,), (0,)), ((), ())),
        preferred_element_type=jnp.float32)

    @pl.when(i == (N_PAD // _BRP) - 1)
    def _():
        pooled = pool_ref[...] / jnp.maximum(cnt_ref[...], 1.0)
        o_ref[...] = jnp.dot(pooled, linw_ref[...],
                             preferred_element_type=jnp.float32) + linb_ref[...]


@jax.jit
def _tc_post(a0, a1, y, dinv, b, batch2d, lin_W, lin_b):
    grid = (N_PAD // _BRP,)
    return pl.pallas_call(
        _tc_post_body,
        grid=grid,
        in_specs=[
            pl.BlockSpec((_BRP, D), lambda i: (i, 0)),
            pl.BlockSpec((_BRP, D), lambda i: (i, 0)),
            pl.BlockSpec((_BRP, D), lambda i: (i, 0)),
            pl.BlockSpec((_BRP, 1), lambda i: (i, 0)),
            pl.BlockSpec((1, D), lambda i: (0, 0)),
            pl.BlockSpec((1, _BRP), lambda i: (0, i)),
            pl.BlockSpec((D, C), lambda i: (0, 0)),
            pl.BlockSpec((1, C), lambda i: (0, 0)),
        ],
        out_specs=pl.BlockSpec((G, C), lambda i: (0, 0)),
        out_shape=jax.ShapeDtypeStruct((G, C), jnp.float32),
        scratch_shapes=[
            pltpu.VMEM((G, D), jnp.float32),
            pltpu.VMEM((G, 1), jnp.float32),
        ],
    )(a0, a1, y, dinv, b, batch2d, lin_W, lin_b)


def kernel(x, edge_index, batch, W1, b1, W2, b2, W3, b3, lin_W, lin_b):
    pad_dst = jnp.broadcast_to(
        N + jnp.arange(EPT - E // NW, dtype=jnp.int32)[None, :],
        (NW, EPT - E // NW))
    dst_r = jnp.concatenate(
        [edge_index[1].astype(jnp.int32).reshape(NW, E // NW), pad_dst],
        axis=1).reshape(NW, CHP, K)
    sd_e = jnp.stack([edge_index[0].astype(jnp.int32).reshape(NW, ECH, EK),
                      edge_index[1].astype(jnp.int32).reshape(NW, ECH, EK)],
                     axis=2)
    x_pad = jnp.pad(x, ((0, N_PAD - N), (0, 0)))
    batch2d = jnp.pad(batch.astype(jnp.int32), (0, N_PAD - N),
                      constant_values=G).reshape(1, N_PAD)

    degp = _sc_deg(dst_r)
    deg0 = degp[0].reshape(N_PAD, 1)
    deg1 = degp[1].reshape(N_PAD, 1)
    dinv, y = _tc_pre(deg0, deg1, x_pad, W1)

    accp = _sc_edge(y, sd_e)
    y = _tc_mid(accp[0], accp[1], y, dinv, b1.reshape(1, D), W2)
    accp = _sc_edge(y, sd_e)
    y = _tc_mid(accp[0], accp[1], y, dinv, b2.reshape(1, D), W3)
    accp = _sc_edge(y, sd_e)
    return _tc_post(accp[0], accp[1], y, dinv, b3.reshape(1, D), batch2d,
                    lin_W, lin_b.reshape(1, C))
